# baseline (speedup 1.0000x reference)
import math

import jax
import jax.numpy as jnp
from jax import lax
from jax.experimental import pallas as pl
from jax.experimental.pallas import tpu as pltpu
from jax.experimental.pallas import tpu_sc as plsc

VOCAB = 1000000
D = 64
B_TOTAL = 16384 * 50
NC, NS = 2, 16
NW = NC * NS
B_PER_W = B_TOTAL // NW
CHUNK = 256
GRP = 128
G = CHUNK // GRP
N_CHUNKS = B_PER_W // CHUNK
NBUF = 4
IDX_ROWS = B_PER_W // GRP
U = 8
SCALE = math.sqrt(D)


def _emb_kernel(w_hbm, idx_hbm, out_hbm, idx_all, rows, *sems):
    gsems = sems[:NBUF]
    osems = sems[NBUF:]
    wid = lax.axis_index("s") * NC + lax.axis_index("c")
    base = wid * B_PER_W
    grow0 = pl.multiple_of(base // GRP, 8)

    pltpu.sync_copy(idx_hbm.at[pl.ds(grow0, IDX_ROWS)], idx_all)

    def fire_gather(g, b):
        for k in range(G):
            pltpu.async_copy(
                w_hbm.at[idx_all.at[g * G + k]],
                rows.at[b, pl.ds(k * GRP, GRP)],
                gsems[b],
            )

    def wait_gather(b):
        pltpu.make_async_copy(
            w_hbm.at[pl.ds(0, CHUNK)], rows.at[b], gsems[b]
        ).wait()

    def scale_chunk(b):
        def body(ri, c):
            for u in range(U):
                r = ri * U + u
                for j in range(D // 16):
                    sl = pl.ds(j * 16, 16)
                    rows[b, r, sl] = rows[b, r, sl] * SCALE
            return c

        lax.fori_loop(0, CHUNK // U, body, 0, unroll=False)

    def fire_out(g, b):
        ooff = pl.multiple_of(base + g * CHUNK, 8)
        pltpu.async_copy(rows.at[b], out_hbm.at[pl.ds(ooff, CHUNK)], osems[b])

    def wait_out(b):
        pltpu.make_async_copy(
            w_hbm.at[pl.ds(0, CHUNK)], rows.at[b], osems[b]
        ).wait()

    fire_gather(0, 0)
    fire_gather(1, 1)

    for g in (0, 1):
        wait_gather(g)
        scale_chunk(g)
        fire_out(g, g)
        fire_gather(g + 2, g + 2)

    @pl.loop(2, N_CHUNKS - 2, step=NBUF)
    def _(go):
        for u in range(NBUF):
            g = go + u
            b = (2 + u) % NBUF
            b2 = u % NBUF
            wait_gather(b)
            scale_chunk(b)
            fire_out(g, b)
            wait_out(b2)
            fire_gather(g + 2, b2)

    for u in range(2):
        g = N_CHUNKS - 2 + u
        b = g % NBUF
        wait_gather(b)
        scale_chunk(b)
        fire_out(g, b)
        wait_out((g + 2) % NBUF)
    wait_out((N_CHUNKS - 2) % NBUF)
    wait_out((N_CHUNKS - 1) % NBUF)


@jax.jit
def _emb(weight, idx2d):
    mesh = plsc.VectorSubcoreMesh(
        core_axis_name="c", subcore_axis_name="s", num_cores=NC, num_subcores=NS
    )
    run = pl.kernel(
        _emb_kernel,
        out_type=jax.ShapeDtypeStruct((B_TOTAL, D), jnp.float32),
        mesh=mesh,
        scratch_types=(
            [
                pltpu.VMEM((IDX_ROWS, GRP), jnp.int32),
                pltpu.VMEM((NBUF, CHUNK, D), jnp.float32),
            ]
            + [pltpu.SemaphoreType.DMA] * (2 * NBUF)
        ),
        compiler_params=pltpu.CompilerParams(use_tc_tiling_on_sc=False),
    )
    return run(weight, idx2d)


def kernel(input_ids, weight):
    idx2d = input_ids.reshape(B_TOTAL // GRP, GRP).astype(jnp.int32)
    out = _emb(weight, idx2d)
    return out.reshape(input_ids.shape + (D,))
